# Initial kernel scaffold; baseline (speedup 1.0000x reference)
import jax
import jax.numpy as jnp
from jax.experimental import pallas as pl


def kernel(x):
    raise NotImplementedError("write your pallas kernel here")


# speedup vs baseline: 13.0794x; 13.0794x over previous
import jax
import jax.numpy as jnp
from jax.experimental import pallas as pl


def _maxes_body(x_ref, diag_ref, off_ref):
    x = x_ref[...]
    m = x.shape[-1]
    row = jax.lax.broadcasted_iota(jnp.int32, (m, m), 0)
    col = jax.lax.broadcasted_iota(jnp.int32, (m, m), 1)
    eq = (row == col)[None]
    neg = jnp.float32(-jnp.inf)
    diag_ref[:, 0, 0] = jnp.max(jnp.where(eq, x, neg), axis=(1, 2))
    off_ref[:, 0, 0] = jnp.max(jnp.where(eq, neg, x), axis=(1, 2))


def kernel(x):
    B, C, m, _ = x.shape
    n_mat = B * C
    xr = x.reshape(n_mat, m, m)
    N = 8
    diag, off = pl.pallas_call(
        _maxes_body,
        grid=(n_mat // N,),
        in_specs=[pl.BlockSpec((N, m, m), lambda i: (i, 0, 0))],
        out_specs=[
            pl.BlockSpec((N, 1, 1), lambda i: (i, 0, 0)),
            pl.BlockSpec((N, 1, 1), lambda i: (i, 0, 0)),
        ],
        out_shape=[jax.ShapeDtypeStruct((n_mat, 1, 1), x.dtype)] * 2,
    )(xr)
    return jnp.concatenate(
        (diag.reshape(B, C), off.reshape(B, C)), axis=-1
    )
